# speedup vs baseline: 1.1576x; 1.1576x over previous
import functools

import jax
import jax.numpy as jnp
from jax import lax
from jax.experimental import pallas as pl
from jax.experimental.pallas import tpu as pltpu
from jax.experimental.pallas import tpu_sc as plsc

B_ROWS = 4096
COLS = 26
D = 128
B = B_ROWS * COLS
NC = 2
NS = 16
NW = NC * NS
B_PER_W = B // NW
C = 128
NCH = B_PER_W // C

_mesh = plsc.VectorSubcoreMesh(core_axis_name="c", subcore_axis_name="s")


@functools.partial(
    pl.kernel,
    mesh=_mesh,
    out_type=jax.ShapeDtypeStruct((B, D), jnp.float32),
    scratch_types=[
        pltpu.VMEM((NCH, C), jnp.int32),
        pltpu.VMEM((C, D), jnp.float32),
        pltpu.SemaphoreType.DMA,
    ],
)
def _gather(table_hbm, idx_hbm, out_hbm, idx_v, rows_v, sem):
    wid = lax.axis_index("s") * NC + lax.axis_index("c")
    base = wid * B_PER_W
    pltpu.sync_copy(idx_hbm.at[wid], idx_v)
    for j in range(NCH):
        pltpu.async_copy(table_hbm.at[idx_v.at[j]], rows_v, sem).wait()
        pltpu.sync_copy(rows_v, out_hbm.at[pl.ds(base + j * C, C)])


def kernel(x, table):
    idx = x.reshape(NW, NCH, C).astype(jnp.int32)
    out = _gather(table, idx)
    return out.reshape(B_ROWS, COLS, D)


# speedup vs baseline: 1.3066x; 1.1287x over previous
import functools

import jax
import jax.numpy as jnp
from jax import lax
from jax.experimental import pallas as pl
from jax.experimental.pallas import tpu as pltpu
from jax.experimental.pallas import tpu_sc as plsc

B_ROWS = 4096
COLS = 26
D = 128
B = B_ROWS * COLS
NC = 2
NS = 16
NW = NC * NS
B_PER_W = B // NW
C = 104
NCH = B_PER_W // C
NBUF = 4

_mesh = plsc.VectorSubcoreMesh(core_axis_name="c", subcore_axis_name="s")


@functools.partial(
    pl.kernel,
    mesh=_mesh,
    out_type=jax.ShapeDtypeStruct((B, D), jnp.float32),
    scratch_types=[
        pltpu.VMEM((NCH, C), jnp.int32),
        pltpu.VMEM((NBUF, C, D), jnp.float32),
    ]
    + [pltpu.SemaphoreType.DMA] * (2 * NBUF),
)
def _gather(table_hbm, idx_hbm, out_hbm, idx_v, rows_v, *sems):
    gsem = sems[:NBUF]
    ssem = sems[NBUF:]
    wid = lax.axis_index("s") * NC + lax.axis_index("c")
    base = wid * B_PER_W
    pltpu.sync_copy(idx_hbm.at[wid], idx_v)

    def gather_start(j, b):
        pltpu.async_copy(table_hbm.at[idx_v.at[j]], rows_v.at[b], gsem[b])

    for b in range(NBUF):
        gather_start(b, b)

    def body(i, carry):
        for b in range(NBUF):
            j = i * NBUF + b
            pltpu.make_async_copy(
                table_hbm.at[idx_v.at[j]], rows_v.at[b], gsem[b]
            ).wait()
            dst = out_hbm.at[pl.ds(base + j * C, C)]
            pltpu.async_copy(rows_v.at[b], dst, ssem[b])

            @pl.when(j + NBUF < NCH)
            def _():
                pltpu.make_async_copy(rows_v.at[b], dst, ssem[b]).wait()
                gather_start(j + NBUF, b)

        return carry

    lax.fori_loop(0, NCH // NBUF, body, 0)
    for b in range(NBUF):
        j = NCH - NBUF + b
        pltpu.make_async_copy(
            rows_v.at[b], out_hbm.at[pl.ds(base + j * C, C)], ssem[b]
        ).wait()


def kernel(x, table):
    idx = x.reshape(NW, NCH, C).astype(jnp.int32)
    out = _gather(table, idx)
    return out.reshape(B_ROWS, COLS, D)


# speedup vs baseline: 2.0592x; 1.5761x over previous
import functools

import jax
import jax.numpy as jnp
from jax import lax
from jax.experimental import pallas as pl
from jax.experimental.pallas import tpu as pltpu
from jax.experimental.pallas import tpu_sc as plsc

B_ROWS = 4096
COLS = 26
D = 128
B = B_ROWS * COLS
NC = 2
NS = 16
NW = NC * NS
B_PER_W = B // NW
RB = 4
C = RB * COLS
NCH = B_PER_W // C
NBUF = 4
ROWS_PER_W = B_ROWS // NW

_mesh = plsc.VectorSubcoreMesh(core_axis_name="c", subcore_axis_name="s")


@functools.partial(
    pl.kernel,
    mesh=_mesh,
    out_type=jax.ShapeDtypeStruct((B_ROWS, COLS, D), jnp.float32),
    scratch_types=[
        pltpu.VMEM((NCH, C), jnp.int32),
        pltpu.VMEM((NBUF, C, D), jnp.float32),
    ]
    + [pltpu.SemaphoreType.DMA] * (2 * NBUF),
    compiler_params=pltpu.CompilerParams(use_tc_tiling_on_sc=True),
)
def _gather(table_hbm, idx_hbm, out_hbm, idx_v, rows_v, *sems):
    gsem = sems[:NBUF]
    ssem = sems[NBUF:]
    wid = lax.axis_index("s") * NC + lax.axis_index("c")
    b0 = wid * ROWS_PER_W
    pltpu.sync_copy(idx_hbm.at[wid], idx_v)

    def gather_start(j, b):
        pltpu.async_copy(table_hbm.at[idx_v.at[j]], rows_v.at[b], gsem[b])

    def store_start(j, b):
        for k in range(RB):
            pltpu.async_copy(
                rows_v.at[b, pl.ds(k * COLS, COLS)],
                out_hbm.at[b0 + j * RB + k],
                ssem[b],
            )

    def store_wait(j, b):
        for k in range(RB):
            pltpu.make_async_copy(
                rows_v.at[b, pl.ds(k * COLS, COLS)],
                out_hbm.at[b0 + j * RB + k],
                ssem[b],
            ).wait()

    for b in range(NBUF):
        gather_start(b, b)

    def body(i, carry):
        for b in range(NBUF):
            j = i * NBUF + b
            pltpu.make_async_copy(
                table_hbm.at[idx_v.at[j]], rows_v.at[b], gsem[b]
            ).wait()
            store_start(j, b)

            @pl.when(j + NBUF < NCH)
            def _():
                store_wait(j, b)
                gather_start(j + NBUF, b)

        return carry

    lax.fori_loop(0, NCH // NBUF, body, 0)
    for b in range(NBUF):
        store_wait(NCH - NBUF + b, b)


def kernel(x, table):
    idx = x.reshape(NW, NCH, C).astype(jnp.int32)
    return _gather(table, idx)


# speedup vs baseline: 3.6704x; 1.7824x over previous
import functools

import jax
import jax.numpy as jnp
from jax import lax
from jax.experimental import pallas as pl
from jax.experimental.pallas import tpu as pltpu
from jax.experimental.pallas import tpu_sc as plsc

B_ROWS = 4096
COLS = 26
D = 128
NC = 2
NS = 16
NW = NC * NS
RPW = B_ROWS // NW
NBUF = 2

_mesh = plsc.VectorSubcoreMesh(core_axis_name="c", subcore_axis_name="s")


@functools.partial(
    pl.kernel,
    mesh=_mesh,
    out_type=jax.ShapeDtypeStruct((COLS, B_ROWS, D), jnp.float32),
    scratch_types=[
        pltpu.VMEM((COLS, RPW), jnp.int32),
        pltpu.VMEM((NBUF, RPW, D), jnp.float32),
    ]
    + [pltpu.SemaphoreType.DMA] * (2 * NBUF),
    compiler_params=pltpu.CompilerParams(use_tc_tiling_on_sc=True),
)
def _gather(table_hbm, idx_hbm, out_hbm, idx_v, rows_v, *sems):
    gsem = sems[:NBUF]
    ssem = sems[NBUF:]
    wid = lax.axis_index("s") * NC + lax.axis_index("c")
    b0 = wid * RPW
    pltpu.sync_copy(idx_hbm.at[wid], idx_v)

    def gather_start(j, b):
        pltpu.async_copy(table_hbm.at[idx_v.at[j]], rows_v.at[b], gsem[b])

    def gather_wait(j, b):
        pltpu.make_async_copy(
            table_hbm.at[idx_v.at[j]], rows_v.at[b], gsem[b]
        ).wait()

    def store_start(j, b):
        pltpu.async_copy(rows_v.at[b], out_hbm.at[j, pl.ds(b0, RPW)], ssem[b])

    def store_wait(j, b):
        pltpu.make_async_copy(
            rows_v.at[b], out_hbm.at[j, pl.ds(b0, RPW)], ssem[b]
        ).wait()

    for b in range(NBUF):
        gather_start(b, b)

    def body(i, carry):
        for b in range(NBUF):
            j = i * NBUF + b
            gather_wait(j, b)
            store_start(j, b)

            @pl.when(j + NBUF < COLS)
            def _():
                store_wait(j, b)
                gather_start(j + NBUF, b)

        return carry

    lax.fori_loop(0, COLS // NBUF, body, 0)
    for b in range(NBUF):
        store_wait(COLS - NBUF + b, b)


def kernel(x, table):
    idx = x.reshape(NW, RPW, COLS).transpose(0, 2, 1).astype(jnp.int32)
    out = _gather(table, idx)
    return jnp.transpose(out, (1, 0, 2))


# speedup vs baseline: 3.7704x; 1.0272x over previous
import functools

import jax
import jax.numpy as jnp
from jax import lax
from jax.experimental import pallas as pl
from jax.experimental.pallas import tpu as pltpu
from jax.experimental.pallas import tpu_sc as plsc

B_ROWS = 4096
COLS = 26
D = 128
NC = 2
NS = 16
NW = NC * NS
RPW = B_ROWS // NW
NBUF = 4
MAIN = (COLS // NBUF) * NBUF

_mesh = plsc.VectorSubcoreMesh(core_axis_name="c", subcore_axis_name="s")


@functools.partial(
    pl.kernel,
    mesh=_mesh,
    out_type=jax.ShapeDtypeStruct((COLS, B_ROWS, D), jnp.float32),
    scratch_types=[
        pltpu.VMEM((COLS, RPW), jnp.int32),
        pltpu.VMEM((NBUF, RPW, D), jnp.float32),
    ]
    + [pltpu.SemaphoreType.DMA] * (2 * NBUF),
    compiler_params=pltpu.CompilerParams(use_tc_tiling_on_sc=True),
)
def _gather(table_hbm, idx_hbm, out_hbm, idx_v, rows_v, *sems):
    gsem = sems[:NBUF]
    ssem = sems[NBUF:]
    wid = lax.axis_index("s") * NC + lax.axis_index("c")
    b0 = wid * RPW
    pltpu.sync_copy(idx_hbm.at[wid], idx_v)

    def gather_start(j, b):
        pltpu.async_copy(table_hbm.at[idx_v.at[j]], rows_v.at[b], gsem[b])

    def gather_wait(j, b):
        pltpu.make_async_copy(
            table_hbm.at[idx_v.at[j]], rows_v.at[b], gsem[b]
        ).wait()

    def store_start(j, b):
        pltpu.async_copy(rows_v.at[b], out_hbm.at[j, pl.ds(b0, RPW)], ssem[b])

    def store_wait(j, b):
        pltpu.make_async_copy(
            rows_v.at[b], out_hbm.at[j, pl.ds(b0, RPW)], ssem[b]
        ).wait()

    for b in range(NBUF):
        gather_start(b, b)

    def body(i, carry):
        for b in range(NBUF):
            j = i * NBUF + b
            gather_wait(j, b)
            store_start(j, b)

            @pl.when(j + NBUF < COLS)
            def _():
                store_wait(j, b)
                gather_start(j + NBUF, b)

        return carry

    lax.fori_loop(0, MAIN // NBUF, body, 0)
    for j in range(MAIN, COLS):
        gather_wait(j, j % NBUF)
        store_start(j, j % NBUF)
    for j in range(COLS - NBUF, COLS):
        store_wait(j, j % NBUF)


def kernel(x, table):
    idx = x.reshape(NW, RPW, COLS).transpose(0, 2, 1).astype(jnp.int32)
    out = _gather(table, idx)
    return jnp.transpose(out, (1, 0, 2))


# speedup vs baseline: 3.8126x; 1.0112x over previous
import functools

import jax
import jax.numpy as jnp
from jax import lax
from jax.experimental import pallas as pl
from jax.experimental.pallas import tpu as pltpu
from jax.experimental.pallas import tpu_sc as plsc

B_ROWS = 4096
COLS = 26
D = 128
NC = 2
NS = 16
NW = NC * NS
RPW = B_ROWS // NW
NBUF = 6
LAG = 3
MAIN = (COLS // NBUF) * NBUF

_mesh = plsc.VectorSubcoreMesh(core_axis_name="c", subcore_axis_name="s")


@functools.partial(
    pl.kernel,
    mesh=_mesh,
    out_type=jax.ShapeDtypeStruct((COLS, B_ROWS, D), jnp.float32),
    scratch_types=[
        pltpu.VMEM((COLS, RPW), jnp.int32),
        pltpu.VMEM((NBUF, RPW, D), jnp.float32),
    ]
    + [pltpu.SemaphoreType.DMA] * (2 * NBUF),
    compiler_params=pltpu.CompilerParams(use_tc_tiling_on_sc=True, skip_device_barrier=True),
)
def _gather(table_hbm, idx_hbm, out_hbm, idx_v, rows_v, *sems):
    gsem = sems[:NBUF]
    ssem = sems[NBUF:]
    wid = lax.axis_index("s") * NC + lax.axis_index("c")
    b0 = wid * RPW
    pltpu.sync_copy(idx_hbm.at[wid], idx_v)

    def gather_start(j, b):
        pltpu.async_copy(table_hbm.at[idx_v.at[j]], rows_v.at[b], gsem[b])

    def gather_wait(j, b):
        pltpu.make_async_copy(
            table_hbm.at[idx_v.at[j]], rows_v.at[b], gsem[b]
        ).wait()

    def store_start(j, b):
        pltpu.async_copy(rows_v.at[b], out_hbm.at[j, pl.ds(b0, RPW)], ssem[b])

    def store_wait(j, b):
        pltpu.make_async_copy(
            rows_v.at[b], out_hbm.at[j, pl.ds(b0, RPW)], ssem[b]
        ).wait()

    def block(j, b):
        gather_wait(j, b)
        store_start(j, b)
        bl = (b - LAG) % NBUF

        @pl.when(j >= LAG)
        def _():
            jl = j - LAG
            store_wait(jl, bl)

            @pl.when(jl + NBUF <= COLS - 1)
            def _():
                gather_start(jl + NBUF, bl)

    for b in range(NBUF):
        gather_start(b, b)

    def body(i, carry):
        for b in range(NBUF):
            block(i * NBUF + b, b)
        return carry

    lax.fori_loop(0, MAIN // NBUF, body, 0)
    for j in range(MAIN, COLS):
        block(j, j % NBUF)
    for j in range(COLS - LAG, COLS):
        store_wait(j, j % NBUF)


def kernel(x, table):
    idx = x.reshape(NW, RPW, COLS).transpose(0, 2, 1).astype(jnp.int32)
    out = _gather(table, idx)
    return jnp.transpose(out, (1, 0, 2))


# speedup vs baseline: 3.8178x; 1.0014x over previous
import functools

import jax
import jax.numpy as jnp
from jax import lax
from jax.experimental import pallas as pl
from jax.experimental.pallas import tpu as pltpu
from jax.experimental.pallas import tpu_sc as plsc

B_ROWS = 4096
COLS = 26
D = 128
NC = 2
NS = 16
NW = NC * NS
RPW = B_ROWS // NW
NBUF = 7
LAG = 3
MAIN = (COLS // NBUF) * NBUF

_mesh = plsc.VectorSubcoreMesh(core_axis_name="c", subcore_axis_name="s")


@functools.partial(
    pl.kernel,
    mesh=_mesh,
    out_type=jax.ShapeDtypeStruct((COLS, B_ROWS, D), jnp.float32),
    scratch_types=[
        pltpu.VMEM((COLS, RPW), jnp.int32),
        pltpu.VMEM((NBUF, RPW, D), jnp.float32),
    ]
    + [pltpu.SemaphoreType.DMA] * (2 * NBUF),
    compiler_params=pltpu.CompilerParams(use_tc_tiling_on_sc=True),
)
def _gather(table_hbm, idx_hbm, out_hbm, idx_v, rows_v, *sems):
    gsem = sems[:NBUF]
    ssem = sems[NBUF:]
    wid = lax.axis_index("s") * NC + lax.axis_index("c")
    b0 = wid * RPW
    pltpu.sync_copy(idx_hbm.at[wid], idx_v)

    def gather_start(j, b):
        pltpu.async_copy(table_hbm.at[idx_v.at[j]], rows_v.at[b], gsem[b])

    def gather_wait(j, b):
        pltpu.make_async_copy(
            table_hbm.at[idx_v.at[j]], rows_v.at[b], gsem[b]
        ).wait()

    def store_start(j, b):
        pltpu.async_copy(rows_v.at[b], out_hbm.at[j, pl.ds(b0, RPW)], ssem[b])

    def store_wait(j, b):
        pltpu.make_async_copy(
            rows_v.at[b], out_hbm.at[j, pl.ds(b0, RPW)], ssem[b]
        ).wait()

    def block(j, b):
        gather_wait(j, b)
        store_start(j, b)
        bl = (b - LAG) % NBUF

        @pl.when(j >= LAG)
        def _():
            jl = j - LAG
            store_wait(jl, bl)

            @pl.when(jl + NBUF <= COLS - 1)
            def _():
                gather_start(jl + NBUF, bl)

    for b in range(NBUF):
        gather_start(b, b)

    def body(i, carry):
        for b in range(NBUF):
            block(i * NBUF + b, b)
        return carry

    lax.fori_loop(0, MAIN // NBUF, body, 0)
    for j in range(MAIN, COLS):
        block(j, j % NBUF)
    for j in range(COLS - LAG, COLS):
        store_wait(j, j % NBUF)


def kernel(x, table):
    idx = x.reshape(NW, RPW, COLS).transpose(0, 2, 1).astype(jnp.int32)
    out = _gather(table, idx)
    return jnp.transpose(out, (1, 0, 2))
